# baseline (speedup 1.0000x reference)
import functools

import jax
import jax.numpy as jnp
from jax import lax
from jax.experimental import pallas as pl
from jax.experimental.pallas import tpu as pltpu
from jax.experimental.pallas import tpu_sc as plsc

_EPS = 1e-12


def _make_sc_fused(V, H, N, S2):
    info = plsc.get_sparse_core_info()
    NC, NS = info.num_cores, info.num_subcores
    NW = NC * NS
    TPW = N // NW
    CH = 16
    NCH = TPW // CH
    NPAIR = NCH // 2
    NF = H // 16
    inv_h = 1.0 / H
    mesh = plsc.VectorSubcoreMesh(core_axis_name="c", subcore_axis_name="s")

    @functools.partial(
        pl.kernel,
        mesh=mesh,
        compiler_params=pltpu.CompilerParams(needs_layout_passes=False),
        out_type=jax.ShapeDtypeStruct((N, H), jnp.float32),
        scratch_types=[
            pltpu.VMEM((TPW,), jnp.int32),
            pltpu.VMEM((TPW,), jnp.int32),
            pltpu.VMEM((CH, H), jnp.float32),
            pltpu.VMEM((CH, H), jnp.float32),
            pltpu.VMEM((CH, H), jnp.float32),
            pltpu.VMEM((CH, H), jnp.float32),
            pltpu.VMEM((CH, H), jnp.float32),
            pltpu.VMEM((CH, H), jnp.float32),
            pltpu.VMEM((H,), jnp.float32),
            pltpu.VMEM((H,), jnp.float32),
            pltpu.SemaphoreType.DMA,
            pltpu.SemaphoreType.DMA,
            pltpu.SemaphoreType.DMA,
            pltpu.SemaphoreType.DMA,
            pltpu.SemaphoreType.DMA,
            pltpu.SemaphoreType.DMA,
        ],
    )
    def fused(table_hbm, pt_hbm, ids_hbm, ptids_hbm, scale_hbm, bias_hbm,
              out_hbm, idx_v, ptidx_v, w0, w1, p0, p1, o0, o1,
              scale_v, bias_v, gw0, gw1, gp0, gp1, so0, so1):
        cid = lax.axis_index("c")
        sid = lax.axis_index("s")
        wid = sid * NC + cid
        base = wid * TPW

        pltpu.sync_copy(ids_hbm.at[pl.ds(base, TPW)], idx_v)
        pltpu.sync_copy(ptids_hbm.at[pl.ds(base, TPW)], ptidx_v)
        pltpu.sync_copy(scale_hbm, scale_v)
        pltpu.sync_copy(bias_hbm, bias_v)

        wbufs = (w0, w1)
        pbufs = (p0, p1)
        obufs = (o0, o1)
        gwsems = (gw0, gw1)
        gpsems = (gp0, gp1)
        ssems = (so0, so1)

        def gather_descs(c, b):
            return (
                pltpu.make_async_copy(
                    table_hbm.at[idx_v.at[pl.ds(c * CH, CH)]],
                    wbufs[b], gwsems[b]),
                pltpu.make_async_copy(
                    pt_hbm.at[ptidx_v.at[pl.ds(c * CH, CH)]],
                    pbufs[b], gpsems[b]),
            )

        def scatter_desc(c, b):
            return pltpu.make_async_copy(
                obufs[b], out_hbm.at[pl.ds(base + c * CH, CH)], ssems[b])

        def ln_token(wb, pb, ob, t):
            s = jnp.zeros((16,), jnp.float32)
            q = jnp.zeros((16,), jnp.float32)
            for f in range(NF):
                x = wb[t, pl.ds(16 * f, 16)] + pb[t, pl.ds(16 * f, 16)]
                ob[t, pl.ds(16 * f, 16)] = x
                s = s + x
                q = q + x * x
            mean = jnp.sum(s) * inv_h
            var = jnp.sum(q) * inv_h - mean * mean + _EPS
            vv = jnp.full((16,), var, jnp.float32)
            iv = plsc.bitcast(vv, jnp.int32)
            iv = 0x5F3759DF - lax.shift_right_arithmetic(iv, 1)
            y = plsc.bitcast(iv, jnp.float32)
            hv = jnp.full((16,), 0.5 * var, jnp.float32)
            for _ in range(3):
                y = y * (1.5 - hv * y * y)
            mv = jnp.full((16,), mean, jnp.float32)
            for f in range(NF):
                x = ob[t, pl.ds(16 * f, 16)]
                xn = (x - mv) * y
                ob[t, pl.ds(16 * f, 16)] = (
                    xn * scale_v[pl.ds(16 * f, 16)] + bias_v[pl.ds(16 * f, 16)])

        def compute(b):
            wb = wbufs[b]
            pb = pbufs[b]
            ob = obufs[b]

            def token_body(tt, carry):
                ln_token(wb, pb, ob, 2 * tt)
                ln_token(wb, pb, ob, 2 * tt + 1)
                return carry

            lax.fori_loop(0, CH // 2, token_body, 0)

        dw, dp = gather_descs(0, 0)
        dw.start()
        dp.start()
        dw, dp = gather_descs(1, 1)
        dw.start()
        dp.start()

        def pair_body(cc, carry):
            for k in range(2):
                c = 2 * cc + k
                dwk, dpk = gather_descs(c, k)
                dwk.wait()
                dpk.wait()

                @pl.when(cc > 0)
                def _():
                    scatter_desc(c - 2, k).wait()

                compute(k)
                scatter_desc(c, k).start()

                @pl.when(cc < NPAIR - 1)
                def _():
                    ndw, ndp = gather_descs(c + 2, k)
                    ndw.start()
                    ndp.start()

            return carry

        lax.fori_loop(0, NPAIR, pair_body, 0)
        scatter_desc(NCH - 2, 0).wait()
        scatter_desc(NCH - 1, 1).wait()

    return fused


def kernel(input_ids, token_type_ids, position_ids, attention_mask,
           word_emb, pos_emb, type_emb, ln_scale, ln_bias):
    B, S = input_ids.shape
    V, H = word_emb.shape
    N = B * S
    ids = input_ids.reshape(N).astype(jnp.int32)
    pt = (type_emb[:, None, :] + pos_emb[None, :, :]).reshape(2 * S, H)
    ptids = (position_ids.reshape(N).astype(jnp.int32)
             + S * token_type_ids.reshape(N).astype(jnp.int32))
    out = _make_sc_fused(V, H, N, 2 * S)(
        word_emb, pt, ids, ptids, ln_scale, ln_bias)
    return out.reshape(B, S, H)
